# Initial kernel scaffold; baseline (speedup 1.0000x reference)
import jax
import jax.numpy as jnp
from jax.experimental import pallas as pl


def kernel(x, table):
    raise NotImplementedError("write your pallas kernel here")


# speedup vs baseline: 1.6837x; 1.6837x over previous
import functools

import jax
import jax.numpy as jnp
from jax import lax
from jax.experimental import pallas as pl
from jax.experimental.pallas import tpu as pltpu
from jax.experimental.pallas import tpu_sc as plsc

NC, NS = 2, 16
NW = NC * NS
B = 16384 * 50
D = 64
CHUNK = 128
PER_W = B // NW
NCHUNK = PER_W // CHUNK

_mesh = plsc.VectorSubcoreMesh(
    core_axis_name="c", subcore_axis_name="s", num_cores=NC, num_subcores=NS
)


@functools.partial(
    pl.kernel,
    out_type=jax.ShapeDtypeStruct((NW, NCHUNK, CHUNK, D), jnp.float32),
    mesh=_mesh,
    scratch_types=[
        pltpu.VMEM((NCHUNK, CHUNK), jnp.int32),
        pltpu.VMEM((CHUNK, D), jnp.float32),
        pltpu.SemaphoreType.DMA,
    ],
    compiler_params=pltpu.CompilerParams(use_tc_tiling_on_sc=False),
)
def _embed_sc(x_hbm, table_hbm, out_hbm, idx_v, rows_v, sem):
    wid = lax.axis_index("s") * NC + lax.axis_index("c")
    pltpu.sync_copy(x_hbm.at[wid], idx_v)

    def step(j, carry):
        pltpu.async_copy(table_hbm.at[idx_v.at[j]], rows_v, sem).wait()
        pltpu.sync_copy(rows_v, out_hbm.at[wid, j])
        return carry

    lax.fori_loop(0, NCHUNK, step, 0, unroll=False)


def kernel(x, table):
    xr = x.reshape(NW, NCHUNK, CHUNK).astype(jnp.int32)
    y = _embed_sc(xr, table)
    return y.reshape(x.shape[0], x.shape[1], D)


# speedup vs baseline: 1.8726x; 1.1122x over previous
import functools

import jax
import jax.numpy as jnp
from jax import lax
from jax.experimental import pallas as pl
from jax.experimental.pallas import tpu as pltpu
from jax.experimental.pallas import tpu_sc as plsc

NC, NS = 2, 16
NW = NC * NS
B = 16384 * 50
D = 64
CHUNK = 128
NBUF = 4
GROUP = NBUF * CHUNK
PER_W = B // NW
NCHUNK = PER_W // CHUNK
NGROUP = PER_W // GROUP

_mesh = plsc.VectorSubcoreMesh(
    core_axis_name="c", subcore_axis_name="s", num_cores=NC, num_subcores=NS
)


@functools.partial(
    pl.kernel,
    out_type=jax.ShapeDtypeStruct((NW, NGROUP, GROUP, D), jnp.float32),
    mesh=_mesh,
    scratch_types=[
        pltpu.VMEM((NCHUNK, CHUNK), jnp.int32),
        pltpu.VMEM((GROUP, D), jnp.float32),
        pltpu.VMEM((GROUP, D), jnp.float32),
        pltpu.SemaphoreType.DMA,
        pltpu.SemaphoreType.DMA,
        pltpu.SemaphoreType.DMA,
        pltpu.SemaphoreType.DMA,
    ],
    compiler_params=pltpu.CompilerParams(use_tc_tiling_on_sc=False),
)
def _embed_sc(x_hbm, table_hbm, out_hbm, idx_v, buf0, buf1,
              gsem0, gsem1, wsem0, wsem1):
    wid = lax.axis_index("s") * NC + lax.axis_index("c")
    pltpu.sync_copy(x_hbm.at[wid], idx_v)

    bufs = (buf0, buf1)
    gsems = (gsem0, gsem1)
    wsems = (wsem0, wsem1)

    def fire_group(g, buf, gsem):
        for b in range(NBUF):
            pltpu.async_copy(
                table_hbm.at[idx_v.at[g * NBUF + b]],
                buf.at[pl.ds(b * CHUNK, CHUNK)],
                gsem,
            )

    fire_group(0, buf0, gsem0)

    @pl.loop(0, NGROUP, step=2)
    def _(g0):
        for p in range(2):
            g = g0 + p
            buf, gsem, wsem = bufs[p], gsems[p], wsems[p]
            obuf, ogsem, owsem = bufs[1 - p], gsems[1 - p], wsems[1 - p]

            pltpu.make_async_copy(
                table_hbm.at[pl.ds(0, GROUP)], buf, gsem
            ).wait()
            pltpu.async_copy(buf, out_hbm.at[wid, g], wsem)

            @pl.when(g + 1 < NGROUP)
            def _():
                @pl.when(g >= 1)
                def _():
                    pltpu.make_async_copy(
                        obuf, out_hbm.at[wid, g - 1], owsem
                    ).wait()
                fire_group(g + 1, obuf, ogsem)

    pltpu.make_async_copy(
        bufs[(NGROUP - 1) % 2],
        out_hbm.at[wid, NGROUP - 1],
        wsems[(NGROUP - 1) % 2],
    ).wait()


def kernel(x, table):
    xr = x.reshape(NW, NCHUNK, CHUNK).astype(jnp.int32)
    y = _embed_sc(xr, table)
    return y.reshape(x.shape[0], x.shape[1], D)
